# Initial kernel scaffold; baseline (speedup 1.0000x reference)
import jax
import jax.numpy as jnp
from jax.experimental import pallas as pl


def kernel(x, weight):
    raise NotImplementedError("write your pallas kernel here")


# speedup vs baseline: 1.7723x; 1.7723x over previous
import functools

import jax
import jax.numpy as jnp
from jax.experimental import pallas as pl
from jax.experimental.pallas import tpu as pltpu


def _mm_kernel(x_ref, w_ref, out_ref):
    out_ref[0] = jax.lax.dot_general(
        w_ref[...], x_ref[0],
        (((1,), (1,)), ((), ())),
        preferred_element_type=jnp.float32,
    )


@jax.jit
def kernel(x, weight):
    B, S, I = x.shape
    O = weight.shape[0]
    S_BLK = min(S, 512)

    grid = (B, S // S_BLK)
    return pl.pallas_call(
        _mm_kernel,
        grid=grid,
        in_specs=[
            pl.BlockSpec((1, S_BLK, I), lambda b, s: (b, s, 0)),
            pl.BlockSpec((O, I), lambda b, s: (0, 0)),
        ],
        out_specs=pl.BlockSpec((1, O, S_BLK), lambda b, s: (b, 0, s)),
        out_shape=jax.ShapeDtypeStruct((B, O, S), jnp.float32),
        compiler_params=pltpu.CompilerParams(
            dimension_semantics=("parallel", "arbitrary"),
        ),
    )(x, weight)
